# baseline (speedup 1.0000x reference)
import functools

import jax
import jax.numpy as jnp
from jax.experimental import pallas as pl

N = 10000
NNZ = 160000
E = 5000
D = 256
TQ = 64

BN = 800
CN = 1250
CE = 1250

f32 = jnp.float32


def _onehot(idx_col, base, width):
    iota = jax.lax.broadcasted_iota(jnp.int32, (1, width), 1) + base
    return (idx_col == iota).astype(f32)


def _dot_t(oh, w):
    return jax.lax.dot_general(oh, w, (((0,), (0,)), ((), ())),
                               preferred_element_type=f32)


def _dot(a, b):
    return jnp.dot(a, b, preferred_element_type=f32)


def _dense_kernel(x_ref, nt_ref, tq_ref, w1t_ref, b1_ref, w2t_ref, b2_ref,
                  wqt_ref, wvt_ref, et_ref, ectx_ref,
                  q_ref, v_ref, at_ref, ec_ref):
    i = pl.program_id(0)
    x = x_ref[...]
    oh3 = (nt_ref[...] == jax.lax.broadcasted_iota(jnp.int32, (1, 3), 1)).astype(f32)
    tsig = _dot(oh3, tq_ref[...])
    h = jnp.concatenate([x, tsig], axis=1)
    pre = jnp.tanh(_dot(h, w1t_ref[...]) + b1_ref[...])
    ts = _dot(pre, w2t_ref[...]) + b2_ref[...]
    at_ref[...] = jax.nn.sigmoid(ts)
    q_ref[...] = _dot(x, wqt_ref[...])
    v_ref[...] = _dot(x, wvt_ref[...])

    @pl.when(i == 0)
    def _():
        ohe = (et_ref[...] == jax.lax.broadcasted_iota(jnp.int32, (1, 3), 1)).astype(f32)
        ec_ref[...] = _dot(ohe, ectx_ref[...])


def _alpha_kernel(nid_ref, eid_ref, q_ref, v_ref, atrow_ref, ec_ref,
                  alpha_ref, vi_ref, segmax_ref):
    i = pl.program_id(0)
    nid = nid_ref[...]
    eid = eid_ref[...]
    qi = jnp.zeros((BN, D), f32)
    vi = jnp.zeros((BN, D), f32)
    ati = jnp.zeros((BN, 1), f32)
    for k in range(N // CN):
        oh = _onehot(nid, k * CN, CN)
        qi = qi + _dot(oh, q_ref[k * CN:(k + 1) * CN, :])
        vi = vi + _dot(oh, v_ref[k * CN:(k + 1) * CN, :])
        ati = ati + jnp.sum(oh * atrow_ref[0:1, k * CN:(k + 1) * CN],
                            axis=1, keepdims=True)
    ctx = jnp.zeros((BN, D), f32)
    for k in range(E // CE):
        ohe = _onehot(eid, k * CE, CE)
        ctx = ctx + _dot(ohe, ec_ref[k * CE:(k + 1) * CE, :])
    an = jnp.sum(qi * ctx, axis=1, keepdims=True)
    an = jnp.where(an >= 0, an, 0.2 * an)
    alpha = an * ati
    alpha_ref[...] = alpha
    vi_ref[...] = vi

    @pl.when(i == 0)
    def _():
        segmax_ref[...] = jnp.full((1, E), -jnp.inf, f32)

    for k in range(E // CE):
        ohe = _onehot(eid, k * CE, CE)
        masked = jnp.where(ohe > 0, alpha, -jnp.inf)
        m = jnp.max(masked, axis=0, keepdims=True)
        sl = pl.ds(k * CE, CE)
        segmax_ref[0:1, sl] = jnp.maximum(segmax_ref[0:1, sl], m)


def _denom_kernel(alpha_ref, eid_ref, segmax_ref, ex_ref, denom_ref):
    i = pl.program_id(0)
    eid = eid_ref[...]
    mi = jnp.zeros((BN, 1), f32)
    for k in range(E // CE):
        ohe = _onehot(eid, k * CE, CE)
        mrow = segmax_ref[0:1, k * CE:(k + 1) * CE]
        mrow = jnp.where(jnp.isfinite(mrow), mrow, 0.0)
        mi = mi + jnp.sum(ohe * mrow, axis=1, keepdims=True)
    ex = jnp.exp(alpha_ref[...] - mi)
    ex_ref[...] = ex

    @pl.when(i == 0)
    def _():
        denom_ref[...] = jnp.zeros((1, E), f32)

    for k in range(E // CE):
        ohe = _onehot(eid, k * CE, CE)
        part = jnp.sum(ohe * ex, axis=0, keepdims=True)
        sl = pl.ds(k * CE, CE)
        denom_ref[0:1, sl] = denom_ref[0:1, sl] + part


def _edge_kernel(ex_ref, eid_ref, denom_ref, vi_ref,
                 attn_ref, ef_ref):
    i = pl.program_id(0)
    eid = eid_ref[...]
    di = jnp.zeros((BN, 1), f32)
    for k in range(E // CE):
        ohe = _onehot(eid, k * CE, CE)
        di = di + jnp.sum(ohe * denom_ref[0:1, k * CE:(k + 1) * CE],
                          axis=1, keepdims=True)
    attn = ex_ref[...] / (di + 1e-16)
    attn_ref[...] = attn
    w = attn * vi_ref[...]

    @pl.when(i == 0)
    def _():
        ef_ref[...] = jnp.zeros((E, D), f32)

    for k in range(E // CE):
        ohe = _onehot(eid, k * CE, CE)
        sl = pl.ds(k * CE, CE)
        ef_ref[sl, :] = ef_ref[sl, :] + _dot_t(ohe, w)


def _node_kernel(attn_ref, eid_ref, nid_ref, ef_ref, nf_ref):
    i = pl.program_id(0)
    eid = eid_ref[...]
    nid = nid_ref[...]
    msg = jnp.zeros((BN, D), f32)
    for k in range(E // CE):
        ohe = _onehot(eid, k * CE, CE)
        msg = msg + _dot(ohe, ef_ref[k * CE:(k + 1) * CE, :])
    w = attn_ref[...] * msg

    @pl.when(i == 0)
    def _():
        nf_ref[...] = jnp.zeros((N, D), f32)

    for k in range(N // CN):
        ohn = _onehot(nid, k * CN, CN)
        sl = pl.ds(k * CN, CN)
        nf_ref[sl, :] = nf_ref[sl, :] + _dot_t(ohn, w)


def _full(shape):
    return pl.BlockSpec(shape, lambda i: (0,) * len(shape))


def _rows(bshape):
    return pl.BlockSpec(bshape, lambda i: (i,) + (0,) * (len(bshape) - 1))


@jax.jit
def kernel(x, node_types, H, edge_type, type_query, W1, b1, W2, b2, Wq, Wv, edge_context):
    nid = H[0].astype(jnp.int32).reshape(NNZ, 1)
    eid = H[1].astype(jnp.int32).reshape(NNZ, 1)
    nt = node_types.astype(jnp.int32).reshape(N, 1)
    et = edge_type.astype(jnp.int32).reshape(E, 1)

    gsteps = NNZ // BN

    q, v, at, ec = pl.pallas_call(
        _dense_kernel,
        grid=(N // BN,),
        in_specs=[_rows((BN, D)), _rows((BN, 1)), _full((3, TQ)),
                  _full((D + TQ, 32)), _full((1, 32)), _full((32, 1)),
                  _full((1, 1)), _full((D, D)), _full((D, D)),
                  _full((E, 1)), _full((3, D))],
        out_specs=[_rows((BN, D)), _rows((BN, D)), _rows((BN, 1)),
                   _full((E, D))],
        out_shape=[jax.ShapeDtypeStruct((N, D), f32),
                   jax.ShapeDtypeStruct((N, D), f32),
                   jax.ShapeDtypeStruct((N, 1), f32),
                   jax.ShapeDtypeStruct((E, D), f32)],
    )(x, nt, type_query, W1.T, b1.reshape(1, 32), W2.T, b2.reshape(1, 1),
      Wq.T, Wv.T, et, edge_context)

    atrow = at.reshape(1, N)

    alpha, vi, segmax = pl.pallas_call(
        _alpha_kernel,
        grid=(gsteps,),
        in_specs=[_rows((BN, 1)), _rows((BN, 1)), _full((N, D)),
                  _full((N, D)), _full((1, N)), _full((E, D))],
        out_specs=[_rows((BN, 1)), _rows((BN, D)), _full((1, E))],
        out_shape=[jax.ShapeDtypeStruct((NNZ, 1), f32),
                   jax.ShapeDtypeStruct((NNZ, D), f32),
                   jax.ShapeDtypeStruct((1, E), f32)],
    )(nid, eid, q, v, atrow, ec)

    ex, denom = pl.pallas_call(
        _denom_kernel,
        grid=(gsteps,),
        in_specs=[_rows((BN, 1)), _rows((BN, 1)), _full((1, E))],
        out_specs=[_rows((BN, 1)), _full((1, E))],
        out_shape=[jax.ShapeDtypeStruct((NNZ, 1), f32),
                   jax.ShapeDtypeStruct((1, E), f32)],
    )(alpha, eid, segmax)

    attn, ef = pl.pallas_call(
        _edge_kernel,
        grid=(gsteps,),
        in_specs=[_rows((BN, 1)), _rows((BN, 1)),
                  _full((1, E)), _rows((BN, D))],
        out_specs=[_rows((BN, 1)), _full((E, D))],
        out_shape=[jax.ShapeDtypeStruct((NNZ, 1), f32),
                   jax.ShapeDtypeStruct((E, D), f32)],
    )(ex, eid, denom, vi)

    nf = pl.pallas_call(
        _node_kernel,
        grid=(gsteps,),
        in_specs=[_rows((BN, 1)), _rows((BN, 1)), _rows((BN, 1)),
                  _full((E, D))],
        out_specs=_full((N, D)),
        out_shape=jax.ShapeDtypeStruct((N, D), f32),
    )(attn, eid, nid, ef)

    return nf, ef


# speedup vs baseline: 1.0006x; 1.0006x over previous
import functools

import jax
import jax.numpy as jnp
from jax.experimental import pallas as pl

N = 10000
NNZ = 160000
E = 5000
D = 256
TQ = 64

BN = 800
BD = 2000
CN = 1250
CE = 1250

f32 = jnp.float32


def _onehot(idx_col, base, width):
    iota = jax.lax.broadcasted_iota(jnp.int32, (1, width), 1) + base
    return (idx_col == iota).astype(f32)


def _dot_t(oh, w):
    return jax.lax.dot_general(oh, w, (((0,), (0,)), ((), ())),
                               preferred_element_type=f32)


def _dot(a, b):
    return jnp.dot(a, b, preferred_element_type=f32)


def _dense_kernel(x_ref, nt_ref, tq_ref, w1t_ref, b1_ref, w2t_ref, b2_ref,
                  wqt_ref, wvt_ref, et_ref, ectx_ref,
                  q_ref, v_ref, at_ref, ec_ref):
    i = pl.program_id(0)
    x = x_ref[...]
    oh3 = (nt_ref[...] == jax.lax.broadcasted_iota(jnp.int32, (1, 3), 1)).astype(f32)
    tsig = _dot(oh3, tq_ref[...])
    h = jnp.concatenate([x, tsig], axis=1)
    pre = jnp.tanh(_dot(h, w1t_ref[...]) + b1_ref[...])
    ts = _dot(pre, w2t_ref[...]) + b2_ref[...]
    at_ref[...] = jax.nn.sigmoid(ts)
    q_ref[...] = _dot(x, wqt_ref[...])
    v_ref[...] = _dot(x, wvt_ref[...])

    @pl.when(i == 0)
    def _():
        ohe = (et_ref[...] == jax.lax.broadcasted_iota(jnp.int32, (1, 3), 1)).astype(f32)
        ec_ref[...] = _dot(ohe, ectx_ref[...])


def _alpha_kernel(nid_ref, eid_ref, q_ref, v_ref, atrow_ref, ec_ref,
                  alpha_ref, vi_ref, segmax_ref):
    i = pl.program_id(0)
    nid = nid_ref[...]
    eid = eid_ref[...]
    qi = jnp.zeros((BN, D), f32)
    vi = jnp.zeros((BN, D), f32)
    ati = jnp.zeros((BN, 1), f32)
    for k in range(N // CN):
        oh = _onehot(nid, k * CN, CN)
        qi = qi + _dot(oh, q_ref[k * CN:(k + 1) * CN, :])
        vi = vi + _dot(oh, v_ref[k * CN:(k + 1) * CN, :])
        ati = ati + jnp.sum(oh * atrow_ref[0:1, k * CN:(k + 1) * CN],
                            axis=1, keepdims=True)
    ctx = jnp.zeros((BN, D), f32)
    for k in range(E // CE):
        ohe = _onehot(eid, k * CE, CE)
        ctx = ctx + _dot(ohe, ec_ref[k * CE:(k + 1) * CE, :])
    an = jnp.sum(qi * ctx, axis=1, keepdims=True)
    an = jnp.where(an >= 0, an, 0.2 * an)
    alpha = an * ati
    alpha_ref[...] = alpha
    vi_ref[...] = vi

    @pl.when(i == 0)
    def _():
        segmax_ref[...] = jnp.full((1, E), -jnp.inf, f32)

    for k in range(E // CE):
        ohe = _onehot(eid, k * CE, CE)
        masked = jnp.where(ohe > 0, alpha, -jnp.inf)
        m = jnp.max(masked, axis=0, keepdims=True)
        sl = pl.ds(k * CE, CE)
        segmax_ref[0:1, sl] = jnp.maximum(segmax_ref[0:1, sl], m)


def _denom_kernel(alpha_ref, eid_ref, segmax_ref, ex_ref, denom_ref):
    i = pl.program_id(0)
    eid = eid_ref[...]
    mi = jnp.zeros((BN, 1), f32)
    for k in range(E // CE):
        ohe = _onehot(eid, k * CE, CE)
        mrow = segmax_ref[0:1, k * CE:(k + 1) * CE]
        mrow = jnp.where(jnp.isfinite(mrow), mrow, 0.0)
        mi = mi + jnp.sum(ohe * mrow, axis=1, keepdims=True)
    ex = jnp.exp(alpha_ref[...] - mi)
    ex_ref[...] = ex

    @pl.when(i == 0)
    def _():
        denom_ref[...] = jnp.zeros((1, E), f32)

    for k in range(E // CE):
        ohe = _onehot(eid, k * CE, CE)
        part = jnp.sum(ohe * ex, axis=0, keepdims=True)
        sl = pl.ds(k * CE, CE)
        denom_ref[0:1, sl] = denom_ref[0:1, sl] + part


def _edge_kernel(ex_ref, eid_ref, denom_ref, vi_ref,
                 attn_ref, ef_ref):
    i = pl.program_id(0)
    eid = eid_ref[...]
    di = jnp.zeros((BN, 1), f32)
    for k in range(E // CE):
        ohe = _onehot(eid, k * CE, CE)
        di = di + jnp.sum(ohe * denom_ref[0:1, k * CE:(k + 1) * CE],
                          axis=1, keepdims=True)
    attn = ex_ref[...] / (di + 1e-16)
    attn_ref[...] = attn
    w = attn * vi_ref[...]

    @pl.when(i == 0)
    def _():
        ef_ref[...] = jnp.zeros((E, D), f32)

    for k in range(E // CE):
        ohe = _onehot(eid, k * CE, CE)
        sl = pl.ds(k * CE, CE)
        ef_ref[sl, :] = ef_ref[sl, :] + _dot_t(ohe, w)


def _node_kernel(attn_ref, eid_ref, nid_ref, ef_ref, nf_ref):
    i = pl.program_id(0)
    eid = eid_ref[...]
    nid = nid_ref[...]
    msg = jnp.zeros((BN, D), f32)
    for k in range(E // CE):
        ohe = _onehot(eid, k * CE, CE)
        msg = msg + _dot(ohe, ef_ref[k * CE:(k + 1) * CE, :])
    w = attn_ref[...] * msg

    @pl.when(i == 0)
    def _():
        nf_ref[...] = jnp.zeros((N, D), f32)

    for k in range(N // CN):
        ohn = _onehot(nid, k * CN, CN)
        sl = pl.ds(k * CN, CN)
        nf_ref[sl, :] = nf_ref[sl, :] + _dot_t(ohn, w)


def _full(shape):
    return pl.BlockSpec(shape, lambda i: (0,) * len(shape))


def _rows(bshape):
    return pl.BlockSpec(bshape, lambda i: (i,) + (0,) * (len(bshape) - 1))


@jax.jit
def kernel(x, node_types, H, edge_type, type_query, W1, b1, W2, b2, Wq, Wv, edge_context):
    nid = H[0].astype(jnp.int32).reshape(NNZ, 1)
    eid = H[1].astype(jnp.int32).reshape(NNZ, 1)
    nt = node_types.astype(jnp.int32).reshape(N, 1)
    et = edge_type.astype(jnp.int32).reshape(E, 1)

    gsteps = NNZ // BN

    q, v, at, ec = pl.pallas_call(
        _dense_kernel,
        grid=(N // BD,),
        in_specs=[_rows((BD, D)), _rows((BD, 1)), _full((3, TQ)),
                  _full((D + TQ, 32)), _full((1, 32)), _full((32, 1)),
                  _full((1, 1)), _full((D, D)), _full((D, D)),
                  _full((E, 1)), _full((3, D))],
        out_specs=[_rows((BD, D)), _rows((BD, D)), _rows((BD, 1)),
                   _full((E, D))],
        out_shape=[jax.ShapeDtypeStruct((N, D), f32),
                   jax.ShapeDtypeStruct((N, D), f32),
                   jax.ShapeDtypeStruct((N, 1), f32),
                   jax.ShapeDtypeStruct((E, D), f32)],
    )(x, nt, type_query, W1.T, b1.reshape(1, 32), W2.T, b2.reshape(1, 1),
      Wq.T, Wv.T, et, edge_context)

    atrow = at.reshape(1, N)

    alpha, vi, segmax = pl.pallas_call(
        _alpha_kernel,
        grid=(gsteps,),
        in_specs=[_rows((BN, 1)), _rows((BN, 1)), _full((N, D)),
                  _full((N, D)), _full((1, N)), _full((E, D))],
        out_specs=[_rows((BN, 1)), _rows((BN, D)), _full((1, E))],
        out_shape=[jax.ShapeDtypeStruct((NNZ, 1), f32),
                   jax.ShapeDtypeStruct((NNZ, D), f32),
                   jax.ShapeDtypeStruct((1, E), f32)],
    )(nid, eid, q, v, atrow, ec)

    ex, denom = pl.pallas_call(
        _denom_kernel,
        grid=(gsteps,),
        in_specs=[_rows((BN, 1)), _rows((BN, 1)), _full((1, E))],
        out_specs=[_rows((BN, 1)), _full((1, E))],
        out_shape=[jax.ShapeDtypeStruct((NNZ, 1), f32),
                   jax.ShapeDtypeStruct((1, E), f32)],
    )(alpha, eid, segmax)

    attn, ef = pl.pallas_call(
        _edge_kernel,
        grid=(gsteps,),
        in_specs=[_rows((BN, 1)), _rows((BN, 1)),
                  _full((1, E)), _rows((BN, D))],
        out_specs=[_rows((BN, 1)), _full((E, D))],
        out_shape=[jax.ShapeDtypeStruct((NNZ, 1), f32),
                   jax.ShapeDtypeStruct((E, D), f32)],
    )(ex, eid, denom, vi)

    nf = pl.pallas_call(
        _node_kernel,
        grid=(gsteps,),
        in_specs=[_rows((BN, 1)), _rows((BN, 1)), _rows((BN, 1)),
                  _full((E, D))],
        out_specs=_full((N, D)),
        out_shape=jax.ShapeDtypeStruct((N, D), f32),
    )(attn, eid, nid, ef)

    return nf, ef
